# baseline (speedup 1.0000x reference)
import functools

import jax
import jax.numpy as jnp
from jax import lax
from jax.experimental import pallas as pl
from jax.experimental.pallas import tpu as pltpu
from jax.experimental.pallas import tpu_sc as plsc

HLEN, WLEN = 128, 128
NBUF = 3
GAHEAD = 2


def _crop_body(ph_ref, pw_ref, bg_ref, out_ref, *, blk_img, c):
    ph = ph_ref[...]
    pw = pw_ref[...]
    for i in range(blk_img):
        for j in range(c):
            out_ref[i, j] = jnp.dot(
                jnp.dot(ph, bg_ref[i, j], preferred_element_type=jnp.float32),
                pw,
                preferred_element_type=jnp.float32,
            )


def _make_spmem_gather(n_img, batch, c):
    rows_per_b = c * HLEN
    info = plsc.get_sparse_core_info()
    nc, ns = info.num_cores, info.num_subcores
    nw = nc * ns
    bpw = batch // nw
    imgs_per_tile = n_img // ns
    mesh = plsc.VectorSubcoreMesh(core_axis_name="c", subcore_axis_name="s")

    @functools.partial(
        pl.kernel,
        out_type=jax.ShapeDtypeStruct((batch * rows_per_b, WLEN), jnp.float32),
        mesh=mesh,
        scratch_types=[
            pltpu.VMEM_SHARED((n_img * HLEN, WLEN), jnp.float32),
            pltpu.VMEM((bpw * HLEN,), jnp.int32),
            pltpu.VMEM((NBUF, HLEN, WLEN), jnp.float32),
            pltpu.SemaphoreType.DMA,
            pltpu.SemaphoreType.DMA,
            pltpu.SemaphoreType.DMA,
            pltpu.SemaphoreType.DMA,
            pltpu.SemaphoreType.DMA,
            pltpu.SemaphoreType.DMA,
        ],
    )
    def sc_gather(table_hbm, spidx_hbm, out_hbm, shared, idx_v,
                  ring, g0, g1, g2, s0, s1, s2):
        cid = lax.axis_index("c")
        sid = lax.axis_index("s")
        wid = sid * nc + cid
        base_b = wid * bpw
        gsems = (g0, g1, g2)
        ssems = (s0, s1, s2)

        pltpu.sync_copy(spidx_hbm.at[pl.ds(base_b * HLEN, bpw * HLEN)], idx_v)

        def gcopy(k):
            p = k % NBUF
            return pltpu.make_async_copy(
                shared.at[idx_v.at[pl.ds((k % bpw) * HLEN, HLEN)]],
                ring.at[p],
                gsems[p],
            )

        def scopy(k):
            p = k % NBUF
            ch, j = divmod(k, bpw)
            b = base_b + j
            return pltpu.make_async_copy(
                ring.at[p],
                out_hbm.at[pl.ds(b * rows_per_b + ch * HLEN, HLEN)],
                ssems[p],
            )

        def gstart(k):
            if k >= NBUF:
                scopy(k - NBUF).wait()
            gcopy(k).start()

        nchunks = c * bpw
        for ch in range(c):
            k0 = ch * bpw
            for t in range(imgs_per_tile):
                img = sid * imgs_per_tile + t
                pltpu.sync_copy(
                    table_hbm.at[pl.ds(img * rows_per_b + ch * HLEN, HLEN)],
                    shared.at[pl.ds(img * HLEN, HLEN)],
                )
            plsc.subcore_barrier()
            for a in range(GAHEAD):
                gstart(k0 + a)
            for j in range(bpw):
                k = k0 + j
                gcopy(k).wait()
                scopy(k).start()
                nk = k + GAHEAD
                if nk < k0 + bpw:
                    gstart(nk)
            plsc.subcore_barrier()
        for k in range(nchunks - NBUF, nchunks):
            scopy(k).wait()

    return sc_gather


def kernel(background, image_id_indices, h, w):
    n_img, c, height, width = background.shape
    batch = image_id_indices.shape[0]
    rows_per_b = c * HLEN

    hh = jnp.asarray(h, jnp.int32)
    ww = jnp.asarray(w, jnp.int32)
    p_h = (
        jnp.arange(height, dtype=jnp.int32)[None, :]
        == hh + jnp.arange(HLEN, dtype=jnp.int32)[:, None]
    ).astype(jnp.float32)
    p_w = (
        jnp.arange(width, dtype=jnp.int32)[:, None]
        == ww + jnp.arange(WLEN, dtype=jnp.int32)[None, :]
    ).astype(jnp.float32)

    blk_img = 16
    crop = pl.pallas_call(
        functools.partial(_crop_body, blk_img=blk_img, c=c),
        grid=(n_img // blk_img,),
        in_specs=[
            pl.BlockSpec((HLEN, height), lambda i: (0, 0)),
            pl.BlockSpec((width, WLEN), lambda i: (0, 0)),
            pl.BlockSpec((blk_img, c, height, width), lambda i: (i, 0, 0, 0)),
        ],
        out_specs=pl.BlockSpec((blk_img, c, HLEN, WLEN), lambda i: (i, 0, 0, 0)),
        out_shape=jax.ShapeDtypeStruct((n_img, c, HLEN, WLEN), background.dtype),
    )
    table = crop(p_h, p_w, background)

    spidx = (
        image_id_indices[:, None] * HLEN + jnp.arange(HLEN, dtype=jnp.int32)
    ).reshape(-1)

    sc_gather = _make_spmem_gather(n_img, batch, c)
    out = sc_gather(table.reshape(n_img * rows_per_b, WLEN), spidx)
    return out.reshape(batch, c, HLEN, WLEN)


# speedup vs baseline: 1.0083x; 1.0083x over previous
import functools

import jax
import jax.numpy as jnp
from jax import lax
from jax.experimental import pallas as pl
from jax.experimental.pallas import tpu as pltpu
from jax.experimental.pallas import tpu_sc as plsc

HLEN, WLEN = 128, 128
NBUF = 3
GAHEAD = 2


def _crop_body(ph_ref, pw_ref, bg_ref, out_ref, *, blk_img, c):
    ph = ph_ref[...]
    pw = pw_ref[...]
    for i in range(blk_img):
        for j in range(c):
            out_ref[i, j] = jnp.dot(
                jnp.dot(ph, bg_ref[i, j], preferred_element_type=jnp.float32),
                pw,
                preferred_element_type=jnp.float32,
            )


def _make_spmem_gather(n_img, batch, c):
    rows_per_b = c * HLEN
    info = plsc.get_sparse_core_info()
    nc, ns = info.num_cores, info.num_subcores
    nw = nc * ns
    bpw = batch // nw
    imgs_per_tile = n_img // ns
    mesh = plsc.VectorSubcoreMesh(core_axis_name="c", subcore_axis_name="s")

    @functools.partial(
        pl.kernel,
        out_type=jax.ShapeDtypeStruct((batch * rows_per_b, WLEN), jnp.float32),
        mesh=mesh,
        scratch_types=[
            pltpu.VMEM_SHARED((n_img * HLEN, WLEN), jnp.float32),
            pltpu.VMEM((bpw * HLEN,), jnp.int32),
            pltpu.VMEM((NBUF, HLEN, WLEN), jnp.float32),
            pltpu.SemaphoreType.DMA,
            pltpu.SemaphoreType.DMA,
            pltpu.SemaphoreType.DMA,
            pltpu.SemaphoreType.DMA,
            pltpu.SemaphoreType.DMA,
            pltpu.SemaphoreType.DMA,
        ],
    )
    def sc_gather(table_hbm, spidx_hbm, out_hbm, shared, idx_v,
                  ring, g0, g1, g2, s0, s1, s2):
        cid = lax.axis_index("c")
        sid = lax.axis_index("s")
        wid = sid * nc + cid
        base_b = wid * bpw
        gsems = (g0, g1, g2)
        ssems = (s0, s1, s2)

        pltpu.sync_copy(spidx_hbm.at[pl.ds(base_b * HLEN, bpw * HLEN)], idx_v)

        def gcopy(k):
            p = k % NBUF
            return pltpu.make_async_copy(
                shared.at[idx_v.at[pl.ds((k % bpw) * HLEN, HLEN)]],
                ring.at[p],
                gsems[p],
            )

        def scopy(k):
            p = k % NBUF
            ch, j = divmod(k, bpw)
            b = base_b + j
            return pltpu.make_async_copy(
                ring.at[p],
                out_hbm.at[pl.ds(b * rows_per_b + ch * HLEN, HLEN)],
                ssems[p],
            )

        def gstart(k):
            if k >= NBUF:
                scopy(k - NBUF).wait()
            gcopy(k).start()

        nchunks = c * bpw
        for ch in range(c):
            k0 = ch * bpw
            for t in range(imgs_per_tile):
                img = sid * imgs_per_tile + t
                pltpu.sync_copy(
                    table_hbm.at[pl.ds(img * rows_per_b + ch * HLEN, HLEN)],
                    shared.at[pl.ds(img * HLEN, HLEN)],
                )
            plsc.subcore_barrier()
            for a in range(GAHEAD):
                gstart(k0 + a)
            for j in range(bpw):
                k = k0 + j
                gcopy(k).wait()
                scopy(k).start()
                nk = k + GAHEAD
                if nk < k0 + bpw:
                    gstart(nk)
            plsc.subcore_barrier()
        for k in range(nchunks - NBUF, nchunks):
            scopy(k).wait()

    return sc_gather


def kernel(background, image_id_indices, h, w):
    n_img, c, height, width = background.shape
    batch = image_id_indices.shape[0]
    rows_per_b = c * HLEN

    hh = jnp.asarray(h, jnp.int32)
    ww = jnp.asarray(w, jnp.int32)
    p_h = (
        jnp.arange(height, dtype=jnp.int32)[None, :]
        == hh + jnp.arange(HLEN, dtype=jnp.int32)[:, None]
    ).astype(jnp.float32)
    p_w = (
        jnp.arange(width, dtype=jnp.int32)[:, None]
        == ww + jnp.arange(WLEN, dtype=jnp.int32)[None, :]
    ).astype(jnp.float32)

    blk_img = 8
    crop = pl.pallas_call(
        functools.partial(_crop_body, blk_img=blk_img, c=c),
        grid=(n_img // blk_img,),
        in_specs=[
            pl.BlockSpec((HLEN, height), lambda i: (0, 0)),
            pl.BlockSpec((width, WLEN), lambda i: (0, 0)),
            pl.BlockSpec((blk_img, c, height, width), lambda i: (i, 0, 0, 0)),
        ],
        out_specs=pl.BlockSpec((blk_img, c, HLEN, WLEN), lambda i: (i, 0, 0, 0)),
        out_shape=jax.ShapeDtypeStruct((n_img, c, HLEN, WLEN), background.dtype),
    )
    table = crop(p_h, p_w, background)

    spidx = (
        image_id_indices[:, None] * HLEN + jnp.arange(HLEN, dtype=jnp.int32)
    ).reshape(-1)

    sc_gather = _make_spmem_gather(n_img, batch, c)
    out = sc_gather(table.reshape(n_img * rows_per_b, WLEN), spidx)
    return out.reshape(batch, c, HLEN, WLEN)
